# baseline (speedup 1.0000x reference)
import functools

import jax
import jax.numpy as jnp
from jax import lax
from jax.experimental import pallas as pl
from jax.experimental.pallas import tpu as pltpu
from jax.experimental.pallas import tpu_sc as plsc

NC = 2
NS = 16
NW = NC * NS
L = 16

B = 64
SEQ = 512
H = 768
HC = H // L
TOK = B * SEQ
TPW = TOK // NW
CH = 32
NCH = TPW // CH
CPR = SEQ // CH
TG = 8
UNROLL = 2
PAD_ID = 1
EPS = 1e-5


def _body(ids_ref, word_ref, pos_ref, tt_ref, g_ref, b_ref, out_ref,
          idx_w0, idx_p0, idx_w1, idx_p1, a0, b0, a1, b1,
          tt_v, g_v, b_v,
          sem_a0, sem_b0, sem_a1, sem_b1, sem_o0, sem_o1):
  wid = lax.axis_index("s") * NC + lax.axis_index("c")
  pltpu.sync_copy(tt_ref, tt_v)
  pltpu.sync_copy(g_ref, g_v)
  pltpu.sync_copy(b_ref, b_v)
  base = wid * TPW

  def tok0_of(c):
    return base + c * CH

  def prep(c, carry_k, idx_w, idx_p):
    pltpu.sync_copy(ids_ref.at[pl.ds(tok0_of(c), CH)], idx_w)
    carry_k = jnp.where(c % CPR == 0, jnp.int32(0), carry_k)

    def pos_loop(j, k):
      ids16 = idx_w[pl.ds(j * L, L)]
      m = jnp.where(ids16 != PAD_ID, jnp.int32(1), jnp.int32(0))
      cs = jnp.cumsum(m) + k
      idx_p[pl.ds(j * L, L)] = cs * m + 1
      return jnp.max(cs)

    return lax.fori_loop(0, CH // L, pos_loop, carry_k)

  def gather_a(idx_w, buf, sem):
    return pltpu.make_async_copy(word_ref.at[idx_w], buf, sem)

  def gather_b(idx_p, buf, sem):
    return pltpu.make_async_copy(pos_ref.at[idx_p], buf, sem)

  def out_copy(c, buf, sem):
    return pltpu.make_async_copy(buf, out_ref.at[pl.ds(tok0_of(c), CH)], sem)

  def ln_chunk(buf_a, buf_b):
    for grp in range(CH // TG):
      t0 = grp * TG

      def p1(j2, carry):
        carry = list(carry)
        for u in range(UNROLL):
          sl = pl.ds((j2 * UNROLL + u) * L, L)
          ttj = tt_v[sl]
          for t in range(TG):
            d = buf_a[t0 + t, sl] + buf_b[t0 + t, sl] + ttj
            buf_a[t0 + t, sl] = d
            carry[2 * t] = carry[2 * t] + d
            carry[2 * t + 1] = carry[2 * t + 1] + d * d
        return tuple(carry)

      z = jnp.zeros((L,), jnp.float32)
      carry = lax.fori_loop(0, HC // UNROLL, p1, (z,) * (2 * TG))

      scales = []
      for t in range(TG):
        mean = jnp.sum(carry[2 * t]) * (1.0 / H)
        ex2 = jnp.sum(carry[2 * t + 1]) * (1.0 / H)
        x = (ex2 - mean * mean) + EPS
        i = lax.bitcast_convert_type(x, jnp.int32)
        i = jnp.int32(0x5F3759DF) - lax.shift_right_logical(i, 1)
        y = lax.bitcast_convert_type(i, jnp.float32)
        y = y * (1.5 - 0.5 * x * y * y)
        y = y * (1.5 - 0.5 * x * y * y)
        y = y * (1.5 - 0.5 * x * y * y)
        scales.append((y, mean * y))

      def p2(j2, _):
        for u in range(UNROLL):
          sl = pl.ds((j2 * UNROLL + u) * L, L)
          gj = g_v[sl]
          bj = b_v[sl]
          for t in range(TG):
            d = buf_a[t0 + t, sl]
            buf_a[t0 + t, sl] = (d * scales[t][0] - scales[t][1]) * gj + bj
        return 0

      lax.fori_loop(0, HC // UNROLL, p2, 0)

  carry_k = prep(0, jnp.int32(0), idx_w0, idx_p0)
  gather_a(idx_w0, a0, sem_a0).start()
  gather_b(idx_p0, b0, sem_b0).start()
  carry_k = prep(1, carry_k, idx_w1, idx_p1)
  gather_a(idx_w1, a1, sem_a1).start()
  gather_b(idx_p1, b1, sem_b1).start()

  def phase(c, carry_k, idx_w, idx_p, buf_a, buf_b, sem_a, sem_b, sem_o):
    gather_a(idx_w, buf_a, sem_a).wait()
    gather_b(idx_p, buf_b, sem_b).wait()
    ln_chunk(buf_a, buf_b)
    out_copy(c, buf_a, sem_o).start()
    c_next = jnp.minimum(c + 2, NCH - 1)
    carry_k = prep(c_next, carry_k, idx_w, idx_p)
    gather_b(idx_p, buf_b, sem_b).start()
    out_copy(c, buf_a, sem_o).wait()
    gather_a(idx_w, buf_a, sem_a).start()
    return carry_k

  def body_i(i, carry_k):
    c = 2 * i
    carry_k = phase(c, carry_k, idx_w0, idx_p0, a0, b0,
                    sem_a0, sem_b0, sem_o0)
    carry_k = phase(c + 1, carry_k, idx_w1, idx_p1, a1, b1,
                    sem_a1, sem_b1, sem_o1)
    return carry_k

  lax.fori_loop(0, NCH // 2, body_i, carry_k)

  gather_a(idx_w0, a0, sem_a0).wait()
  gather_b(idx_p0, b0, sem_b0).wait()
  gather_a(idx_w1, a1, sem_a1).wait()
  gather_b(idx_p1, b1, sem_b1).wait()


@functools.partial(
    pl.kernel,
    out_type=jax.ShapeDtypeStruct((TOK, H), jnp.float32),
    mesh=plsc.VectorSubcoreMesh(
        core_axis_name="c", subcore_axis_name="s",
        num_cores=NC, num_subcores=NS),
    compiler_params=pltpu.CompilerParams(needs_layout_passes=False),
    scratch_types=[
        pltpu.VMEM((CH,), jnp.int32),
        pltpu.VMEM((CH,), jnp.int32),
        pltpu.VMEM((CH,), jnp.int32),
        pltpu.VMEM((CH,), jnp.int32),
        pltpu.VMEM((CH, H), jnp.float32),
        pltpu.VMEM((CH, H), jnp.float32),
        pltpu.VMEM((CH, H), jnp.float32),
        pltpu.VMEM((CH, H), jnp.float32),
        pltpu.VMEM((H,), jnp.float32),
        pltpu.VMEM((H,), jnp.float32),
        pltpu.VMEM((H,), jnp.float32),
        pltpu.SemaphoreType.DMA,
        pltpu.SemaphoreType.DMA,
        pltpu.SemaphoreType.DMA,
        pltpu.SemaphoreType.DMA,
        pltpu.SemaphoreType.DMA,
        pltpu.SemaphoreType.DMA,
    ],
)
def _sc_embed_ln(ids_ref, word_ref, pos_ref, tt_ref, g_ref, b_ref, out_ref,
                 idx_w0, idx_p0, idx_w1, idx_p1, a0, b0, a1, b1,
                 tt_v, g_v, b_v,
                 sem_a0, sem_b0, sem_a1, sem_b1, sem_o0, sem_o1):
  _body(ids_ref, word_ref, pos_ref, tt_ref, g_ref, b_ref, out_ref,
        idx_w0, idx_p0, idx_w1, idx_p1, a0, b0, a1, b1,
        tt_v, g_v, b_v,
        sem_a0, sem_b0, sem_a1, sem_b1, sem_o0, sem_o1)


@jax.jit
def kernel(input_ids, word_embeddings, token_type_embeddings,
           position_embeddings, ln_gamma, ln_beta):
  ids = input_ids.reshape(TOK).astype(jnp.int32)
  tt_row = token_type_embeddings.reshape(H)
  out = _sc_embed_ln(ids, word_embeddings, position_embeddings,
                     tt_row, ln_gamma, ln_beta)
  return out.reshape(B, SEQ, H)


# speedup vs baseline: 2.8810x; 2.8810x over previous
import functools

import jax
import jax.numpy as jnp
from jax import lax
from jax.experimental import pallas as pl
from jax.experimental.pallas import tpu as pltpu
from jax.experimental.pallas import tpu_sc as plsc

NC = 2
NS = 16
NW = NC * NS
L = 16

B = 64
SEQ = 512
H = 768
HC = H // L
TOK = B * SEQ
TPW = TOK // NW
CH = 32
NCH = TPW // CH
CPR = SEQ // CH
TG = 8
UNROLL = 1
PAD_ID = 1
EPS = 1e-5


def _body(ids_ref, word_ref, pos_ref, tt_ref, g_ref, b_ref, out_ref,
          idx_w0, idx_p0, idx_w1, idx_p1, a0, b0, a1, b1,
          tt_v, g_v, b_v,
          sem_a0, sem_b0, sem_a1, sem_b1, sem_o0, sem_o1):
  wid = lax.axis_index("s") * NC + lax.axis_index("c")
  pltpu.sync_copy(tt_ref, tt_v)
  pltpu.sync_copy(g_ref, g_v)
  pltpu.sync_copy(b_ref, b_v)
  base = wid * TPW

  def tok0_of(c):
    return base + c * CH

  def prep(c, carry_k, idx_w, idx_p):
    pltpu.sync_copy(ids_ref.at[pl.ds(tok0_of(c), CH)], idx_w)
    carry_k = jnp.where(c % CPR == 0, jnp.int32(0), carry_k)

    def pos_loop(j, k):
      ids16 = idx_w[pl.ds(j * L, L)]
      m = jnp.where(ids16 != PAD_ID, jnp.int32(1), jnp.int32(0))
      cs = jnp.cumsum(m) + k
      idx_p[pl.ds(j * L, L)] = cs * m + 1
      return jnp.max(cs)

    return lax.fori_loop(0, CH // L, pos_loop, carry_k)

  def gather_a(idx_w, buf, sem):
    return pltpu.make_async_copy(word_ref.at[idx_w], buf, sem)

  def gather_b(idx_p, buf, sem):
    return pltpu.make_async_copy(pos_ref.at[idx_p], buf, sem)

  def out_copy(c, buf, sem):
    return pltpu.make_async_copy(buf, out_ref.at[pl.ds(tok0_of(c), CH)], sem)

  def ln_chunk(buf_a, buf_b):
    for grp in range(CH // TG):
      t0 = grp * TG

      def p1(j2, carry):
        carry = list(carry)
        for u in range(UNROLL):
          sl = pl.ds((j2 * UNROLL + u) * L, L)
          ttj = tt_v[sl]
          for t in range(TG):
            d = buf_a[t0 + t, sl] + buf_b[t0 + t, sl] + ttj
            buf_a[t0 + t, sl] = d
            carry[2 * t] = carry[2 * t] + d
            carry[2 * t + 1] = carry[2 * t + 1] + d * d
        return tuple(carry)

      z = jnp.zeros((L,), jnp.float32)
      carry = lax.fori_loop(0, HC // UNROLL, p1, (z,) * (2 * TG))

      scales = []
      for t in range(TG):
        mean = jnp.sum(carry[2 * t]) * (1.0 / H)
        ex2 = jnp.sum(carry[2 * t + 1]) * (1.0 / H)
        x = (ex2 - mean * mean) + EPS
        i = lax.bitcast_convert_type(x, jnp.int32)
        i = jnp.int32(0x5F3759DF) - lax.shift_right_logical(i, 1)
        y = lax.bitcast_convert_type(i, jnp.float32)
        y = y * (1.5 - 0.5 * x * y * y)
        y = y * (1.5 - 0.5 * x * y * y)
        y = y * (1.5 - 0.5 * x * y * y)
        scales.append((y, mean * y))

      def p2(j2, _):
        for u in range(UNROLL):
          sl = pl.ds((j2 * UNROLL + u) * L, L)
          gj = g_v[sl]
          bj = b_v[sl]
          for t in range(TG):
            d = buf_a[t0 + t, sl]
            buf_a[t0 + t, sl] = (d * scales[t][0] - scales[t][1]) * gj + bj
        return 0

      lax.fori_loop(0, HC // UNROLL, p2, 0)

  carry_k = prep(0, jnp.int32(0), idx_w0, idx_p0)
  gather_a(idx_w0, a0, sem_a0).start()
  gather_b(idx_p0, b0, sem_b0).start()
  carry_k = prep(1, carry_k, idx_w1, idx_p1)
  gather_a(idx_w1, a1, sem_a1).start()
  gather_b(idx_p1, b1, sem_b1).start()

  def phase(c, carry_k, idx_w, idx_p, buf_a, buf_b, sem_a, sem_b, sem_o):
    gather_a(idx_w, buf_a, sem_a).wait()
    gather_b(idx_p, buf_b, sem_b).wait()
    ln_chunk(buf_a, buf_b)
    out_copy(c, buf_a, sem_o).start()
    c_next = jnp.minimum(c + 2, NCH - 1)
    carry_k = prep(c_next, carry_k, idx_w, idx_p)
    gather_b(idx_p, buf_b, sem_b).start()
    out_copy(c, buf_a, sem_o).wait()
    gather_a(idx_w, buf_a, sem_a).start()
    return carry_k

  def body_i(i, carry_k):
    c = 2 * i
    carry_k = phase(c, carry_k, idx_w0, idx_p0, a0, b0,
                    sem_a0, sem_b0, sem_o0)
    carry_k = phase(c + 1, carry_k, idx_w1, idx_p1, a1, b1,
                    sem_a1, sem_b1, sem_o1)
    return carry_k

  lax.fori_loop(0, NCH // 2, body_i, carry_k)

  gather_a(idx_w0, a0, sem_a0).wait()
  gather_b(idx_p0, b0, sem_b0).wait()
  gather_a(idx_w1, a1, sem_a1).wait()
  gather_b(idx_p1, b1, sem_b1).wait()


@functools.partial(
    pl.kernel,
    out_type=jax.ShapeDtypeStruct((TOK, H), jnp.float32),
    mesh=plsc.VectorSubcoreMesh(
        core_axis_name="c", subcore_axis_name="s",
        num_cores=NC, num_subcores=NS),
    compiler_params=pltpu.CompilerParams(needs_layout_passes=False),
    scratch_types=[
        pltpu.VMEM((CH,), jnp.int32),
        pltpu.VMEM((CH,), jnp.int32),
        pltpu.VMEM((CH,), jnp.int32),
        pltpu.VMEM((CH,), jnp.int32),
        pltpu.VMEM((CH, H), jnp.float32),
        pltpu.VMEM((CH, H), jnp.float32),
        pltpu.VMEM((CH, H), jnp.float32),
        pltpu.VMEM((CH, H), jnp.float32),
        pltpu.VMEM((H,), jnp.float32),
        pltpu.VMEM((H,), jnp.float32),
        pltpu.VMEM((H,), jnp.float32),
        pltpu.SemaphoreType.DMA,
        pltpu.SemaphoreType.DMA,
        pltpu.SemaphoreType.DMA,
        pltpu.SemaphoreType.DMA,
        pltpu.SemaphoreType.DMA,
        pltpu.SemaphoreType.DMA,
    ],
)
def _sc_embed_ln(ids_ref, word_ref, pos_ref, tt_ref, g_ref, b_ref, out_ref,
                 idx_w0, idx_p0, idx_w1, idx_p1, a0, b0, a1, b1,
                 tt_v, g_v, b_v,
                 sem_a0, sem_b0, sem_a1, sem_b1, sem_o0, sem_o1):
  _body(ids_ref, word_ref, pos_ref, tt_ref, g_ref, b_ref, out_ref,
        idx_w0, idx_p0, idx_w1, idx_p1, a0, b0, a1, b1,
        tt_v, g_v, b_v,
        sem_a0, sem_b0, sem_a1, sem_b1, sem_o0, sem_o1)


@jax.jit
def kernel(input_ids, word_embeddings, token_type_embeddings,
           position_embeddings, ln_gamma, ln_beta):
  ids = input_ids.reshape(TOK).astype(jnp.int32)
  tt_row = token_type_embeddings.reshape(H)
  out = _sc_embed_ln(ids, word_embeddings, position_embeddings,
                     tt_row, ln_gamma, ln_beta)
  return out.reshape(B, SEQ, H)
